# baseline (speedup 1.0000x reference)
import functools

import jax
import jax.numpy as jnp
from jax import lax
from jax.experimental import pallas as pl
from jax.experimental.pallas import tpu as pltpu
from jax.experimental.pallas import tpu_sc as plsc

_NC, _NS = 2, 16
_NW = _NC * _NS


def _make_sc_copy(max_len, d_model, chunk):
    rows_per_w = max_len // _NW
    nchunk = rows_per_w // chunk
    mesh = plsc.VectorSubcoreMesh(core_axis_name="c", subcore_axis_name="s")

    @functools.partial(
        pl.kernel,
        out_type=jax.ShapeDtypeStruct((max_len, d_model), jnp.float32),
        mesh=mesh,
        scratch_types=[pltpu.VMEM((chunk, d_model), jnp.float32)],
    )
    def sc_copy(emb_hbm, out_hbm, buf):
        wid = lax.axis_index("s") * _NC + lax.axis_index("c")
        base = wid * rows_per_w
        for k in range(nchunk):
            pltpu.sync_copy(emb_hbm.at[pl.ds(base + k * chunk, chunk)], buf)
            pltpu.sync_copy(buf, out_hbm.at[pl.ds(base + k * chunk, chunk)])

    return sc_copy


def kernel(x, embedding):
    seq_len = x.shape[1]
    max_len, d_model = embedding.shape
    sc_copy = _make_sc_copy(max_len, d_model, chunk=128)
    out = sc_copy(embedding)
    return out[None, :seq_len, :]
